# baseline (speedup 1.0000x reference)
import jax
import jax.numpy as jnp
from jax.experimental import pallas as pl
from jax.experimental.pallas import tpu as pltpu

N = 100000
D = 128
B = 64
STEPS = 3
BN = 20000
NBLK = N // BN
NC = 2
CN = BN // NC
NEG = -1e30


def _body(xb_ref, batcht_ref, wih_ref, whh_ref, b_ref, out_ref,
          h_ref, c_ref, qs_ref, m_ref, s_ref, r_ref):
    step = pl.program_id(0)
    blk = pl.program_id(1)

    @pl.when(blk == 0)
    def _start_step():
        @pl.when(step == 0)
        def _init():
            qs_ref[...] = jnp.zeros((B, 2 * D), jnp.float32)
            h_ref[...] = jnp.zeros((B, D), jnp.float32)
            c_ref[...] = jnp.zeros((B, D), jnp.float32)

        @pl.when(step > 0)
        def _finalize_prev():
            s = s_ref[...]
            denom = jnp.where(s > 0.0, s, 1.0)
            qs_ref[:, D:] = r_ref[...] / denom
            qs_ref[:, :D] = h_ref[...]

        gates = (
            jnp.dot(qs_ref[...], wih_ref[...], preferred_element_type=jnp.float32)
            + jnp.dot(h_ref[...], whh_ref[...], preferred_element_type=jnp.float32)
            + b_ref[...]
        )
        i_g = jax.nn.sigmoid(gates[:, :D])
        f_g = jax.nn.sigmoid(gates[:, D:2 * D])
        g_g = jnp.tanh(gates[:, 2 * D:3 * D])
        o_g = jax.nn.sigmoid(gates[:, 3 * D:])
        c = f_g * c_ref[...] + i_g * g_g
        c_ref[...] = c
        h_ref[...] = o_g * jnp.tanh(c)

        m_ref[...] = jnp.zeros((B, 1), jnp.float32)
        s_ref[...] = jnp.zeros((B, 1), jnp.float32)
        r_ref[...] = jnp.zeros((B, D), jnp.float32)

    q = h_ref[...].astype(jnp.bfloat16)
    row = jax.lax.broadcasted_iota(jnp.int32, (B, 1), 0)

    xbs = [
        xb_ref[ci * CN:(ci + 1) * CN, :].astype(jnp.bfloat16) for ci in range(NC)
    ]
    es = []
    for ci in range(NC):
        scores = jax.lax.dot_general(
            q, xbs[ci], (((1,), (1,)), ((), ())), preferred_element_type=jnp.float32
        )
        seg = batcht_ref[0, ci:ci + 1, :]
        es.append(jnp.where(seg == row, scores, NEG))

    m_old = m_ref[...]
    m_new = m_old
    for e in es:
        m_new = jnp.maximum(m_new, jnp.max(e, axis=1, keepdims=True))
    scale = jnp.exp(m_old - m_new)

    pr = jnp.zeros((B, D), jnp.float32)
    s_add = jnp.zeros((B, 1), jnp.float32)
    for ci, e in enumerate(es):
        p = jnp.exp(e - m_new)
        pb = p.astype(jnp.bfloat16)
        pr = pr + jax.lax.dot_general(
            pb, xbs[ci], (((1,), (0,)), ((), ())), preferred_element_type=jnp.float32
        )
        s_add = s_add + jnp.sum(p, axis=1, keepdims=True)

    r_ref[...] = r_ref[...] * scale + pr
    s_ref[...] = s_ref[...] * scale + s_add
    m_ref[...] = m_new

    @pl.when(jnp.logical_and(step == STEPS - 1, blk == NBLK - 1))
    def _emit():
        s = s_ref[...]
        denom = jnp.where(s > 0.0, s, 1.0)
        out_ref[:, :D] = h_ref[...]
        out_ref[:, D:] = r_ref[...] / denom


def kernel(x, batch, W_ih, W_hh, b_ih, b_hh):
    batcht = batch.astype(jnp.int32).reshape(NBLK, NC, CN)
    bias = (b_ih + b_hh).reshape(1, 4 * D)
    wih_t = W_ih.T
    whh_t = W_hh.T

    return pl.pallas_call(
        _body,
        grid=(STEPS, NBLK),
        in_specs=[
            pl.BlockSpec((BN, D), lambda s, k: (k, 0)),
            pl.BlockSpec((1, NC, CN), lambda s, k: (k, 0, 0)),
            pl.BlockSpec((2 * D, 4 * D), lambda s, k: (0, 0)),
            pl.BlockSpec((D, 4 * D), lambda s, k: (0, 0)),
            pl.BlockSpec((1, 4 * D), lambda s, k: (0, 0)),
        ],
        out_specs=pl.BlockSpec((B, 2 * D), lambda s, k: (0, 0)),
        out_shape=jax.ShapeDtypeStruct((B, 2 * D), jnp.float32),
        scratch_shapes=[
            pltpu.VMEM((B, D), jnp.float32),
            pltpu.VMEM((B, D), jnp.float32),
            pltpu.VMEM((B, 2 * D), jnp.float32),
            pltpu.VMEM((B, 1), jnp.float32),
            pltpu.VMEM((B, 1), jnp.float32),
            pltpu.VMEM((B, D), jnp.float32),
        ],
    )(x, batcht, wih_t, whh_t, bias)


# speedup vs baseline: 1.0020x; 1.0020x over previous
import jax
import jax.numpy as jnp
from jax.experimental import pallas as pl
from jax.experimental.pallas import tpu as pltpu

N = 100000
D = 128
B = 64
STEPS = 3
BN = 20000
NBLK = N // BN
NC = 2
CN = BN // NC
NEG = -1e30


def _body(xb_ref, batcht_ref, wih_ref, whh_ref, b_ref, out_ref,
          h_ref, c_ref, qs_ref, m_ref, s_ref, r_ref):
    step = pl.program_id(0)
    blk = pl.program_id(1)

    @pl.when(blk == 0)
    def _start_step():
        @pl.when(step == 0)
        def _init():
            qs_ref[...] = jnp.zeros((B, 2 * D), jnp.float32)
            h_ref[...] = jnp.zeros((B, D), jnp.float32)
            c_ref[...] = jnp.zeros((B, D), jnp.float32)

        @pl.when(step > 0)
        def _finalize_prev():
            s = s_ref[...]
            denom = jnp.where(s > 0.0, s, 1.0)
            qs_ref[:, D:] = r_ref[...] / denom
            qs_ref[:, :D] = h_ref[...]

        gates = (
            jnp.dot(qs_ref[...], wih_ref[...], preferred_element_type=jnp.float32)
            + jnp.dot(h_ref[...], whh_ref[...], preferred_element_type=jnp.float32)
            + b_ref[...]
        )
        i_g = jax.nn.sigmoid(gates[:, :D])
        f_g = jax.nn.sigmoid(gates[:, D:2 * D])
        g_g = jnp.tanh(gates[:, 2 * D:3 * D])
        o_g = jax.nn.sigmoid(gates[:, 3 * D:])
        c = f_g * c_ref[...] + i_g * g_g
        c_ref[...] = c
        h_ref[...] = o_g * jnp.tanh(c)

        m_ref[...] = jnp.zeros((B, 1), jnp.float32)
        s_ref[...] = jnp.zeros((B, 1), jnp.float32)
        r_ref[...] = jnp.zeros((B, D), jnp.float32)

    q = h_ref[...].astype(jnp.bfloat16)
    row = jax.lax.broadcasted_iota(jnp.int32, (B, 1), 0).astype(jnp.bfloat16)

    xbs = [
        xb_ref[ci * CN:(ci + 1) * CN, :].astype(jnp.bfloat16) for ci in range(NC)
    ]
    es = []
    for ci in range(NC):
        scores = jax.lax.dot_general(
            q, xbs[ci], (((1,), (1,)), ((), ())),
            preferred_element_type=jnp.float32,
        ).astype(jnp.bfloat16)
        seg = batcht_ref[0, ci:ci + 1, :]
        es.append(jnp.where(seg == row, scores, jnp.bfloat16(NEG)))

    m_old = m_ref[...]
    m_new = m_old
    for e in es:
        m_new = jnp.maximum(m_new, jnp.max(e, axis=1, keepdims=True).astype(jnp.float32))
    scale = jnp.exp(m_old - m_new)
    m_newb = m_new.astype(jnp.bfloat16)

    pr = jnp.zeros((B, D), jnp.float32)
    s_add = jnp.zeros((B, 1), jnp.float32)
    ones = jnp.ones((CN, 8), jnp.bfloat16)
    for ci, e in enumerate(es):
        pb = jnp.exp(e - m_newb)
        pr = pr + jax.lax.dot_general(
            pb, xbs[ci], (((1,), (0,)), ((), ())), preferred_element_type=jnp.float32
        )
        s8 = jax.lax.dot_general(
            pb, ones, (((1,), (0,)), ((), ())), preferred_element_type=jnp.float32
        )
        s_add = s_add + s8[:, 0:1]

    r_ref[...] = r_ref[...] * scale + pr
    s_ref[...] = s_ref[...] * scale + s_add
    m_ref[...] = m_new

    @pl.when(jnp.logical_and(step == STEPS - 1, blk == NBLK - 1))
    def _emit():
        s = s_ref[...]
        denom = jnp.where(s > 0.0, s, 1.0)
        out_ref[:, :D] = h_ref[...]
        out_ref[:, D:] = r_ref[...] / denom


def kernel(x, batch, W_ih, W_hh, b_ih, b_hh):
    batcht = batch.astype(jnp.bfloat16).reshape(NBLK, NC, CN)
    bias = (b_ih + b_hh).reshape(1, 4 * D)
    wih_t = W_ih.T
    whh_t = W_hh.T

    return pl.pallas_call(
        _body,
        grid=(STEPS, NBLK),
        in_specs=[
            pl.BlockSpec((BN, D), lambda s, k: (k, 0)),
            pl.BlockSpec((1, NC, CN), lambda s, k: (k, 0, 0)),
            pl.BlockSpec((2 * D, 4 * D), lambda s, k: (0, 0)),
            pl.BlockSpec((D, 4 * D), lambda s, k: (0, 0)),
            pl.BlockSpec((1, 4 * D), lambda s, k: (0, 0)),
        ],
        out_specs=pl.BlockSpec((B, 2 * D), lambda s, k: (0, 0)),
        out_shape=jax.ShapeDtypeStruct((B, 2 * D), jnp.float32),
        scratch_shapes=[
            pltpu.VMEM((B, D), jnp.float32),
            pltpu.VMEM((B, D), jnp.float32),
            pltpu.VMEM((B, 2 * D), jnp.float32),
            pltpu.VMEM((B, 1), jnp.float32),
            pltpu.VMEM((B, 1), jnp.float32),
            pltpu.VMEM((B, D), jnp.float32),
        ],
    )(x, batcht, wih_t, whh_t, bias)
